# baseline (speedup 1.0000x reference)
import functools

import jax
import jax.numpy as jnp
from jax import lax
from jax.experimental import pallas as pl
from jax.experimental.pallas import tpu as pltpu
from jax.experimental.pallas import tpu_sc as plsc

D = 64
B = 16384
NC = 2
NS = 16
NW = NC * NS
PER_W = B // NW
CH = 128
NCH = PER_W // CH

_GATHERS = (
    (0, 0),
    (0, 1),
    (1, 2),
    (2, 2),
    (3, 3),
    (4, 4),
    (5, 5),
    (0, 6),
    (0, 7),
    (1, 8),
    (2, 8),
    (3, 9),
    (4, 10),
    (5, 11),
    (0, 12),
    (1, 13),
    (2, 13),
)
NG = len(_GATHERS)
NIDX = 14

_MESH = plsc.VectorSubcoreMesh(
    core_axis_name="c", subcore_axis_name="s", num_cores=NC, num_subcores=NS
)


@functools.partial(
    pl.kernel,
    out_type=jax.ShapeDtypeStruct((NG, B, D), jnp.float32),
    mesh=_MESH,
    scratch_types=[
        pltpu.VMEM((NIDX, PER_W), jnp.int32),
        pltpu.VMEM((CH, D), jnp.float32),
        pltpu.SemaphoreType.DMA,
    ],
    compiler_params=pltpu.CompilerParams(use_tc_tiling_on_sc=False),
)
def _sc_gather(ent, bch, bct, head, tail, rel, idx_hbm, out, idx_v, rows, sem):
    tables = (ent, bch, bct, head, tail, rel)
    wid = lax.axis_index("s") * NC + lax.axis_index("c")
    base = wid * PER_W
    pltpu.async_copy(idx_hbm.at[:, pl.ds(base, PER_W)], idx_v, sem).wait()
    for g, (ti, ii) in enumerate(_GATHERS):
        tbl = tables[ti]

        @pl.loop(0, NCH)
        def _(c, g=g, tbl=tbl, ii=ii):
            off = c * CH
            pltpu.sync_copy(tbl.at[idx_v.at[ii, pl.ds(off, CH)]], rows)
            pltpu.sync_copy(rows, out.at[g, pl.ds(base + off, CH)])


_R = 2048


def _tc_body(m_ref, g_ref, o1, o2, o3, o4, o5, o6):
    m = m_ref[0, 0]
    one = jnp.float32(1.0)
    x = g_ref[...]
    aBHEE, aBTEE, aBCHE, aBCTE = x[0], x[1], x[2], x[3]
    aHeadE, aTailE, aRelE = x[4], x[5], x[6]
    nABHEE, nABTEE, nABCHE, nABCTE = x[7], x[8], x[9], x[10]
    nHeadE, nTailE, nRelE = x[11], x[12], x[13]
    uniqEE, uniqBCHE, uniqBCTE = x[14], x[15], x[16]

    tmpBE2CH = (one - aBCHE) * aBHEE
    tmpBE2CT = (one - aBCTE) * aBTEE
    tmpTransE = jnp.sum(jnp.abs(aHeadE + aRelE - aTailE), axis=1)
    o1[...] = (jnp.sum(tmpBE2CH * tmpBE2CH, axis=1)
               + jnp.sum(tmpBE2CT * tmpBE2CT, axis=1) + tmpTransE)
    tmpNBE2CH = (one - nABCHE) * nABHEE
    tmpNBE2CT = (one - nABCTE) * nABTEE
    tmpNTransE = jnp.sum(jnp.abs(nHeadE + nRelE - nTailE), axis=1)
    tmpNBL = (jnp.sum(tmpNBE2CH * tmpNBE2CH, axis=1)
              + jnp.sum(tmpNBE2CT * tmpNBE2CT, axis=1) + tmpNTransE)
    o2[...] = jnp.maximum(m - tmpNBL, 0.0)
    tmpE = jnp.sum(uniqEE * uniqEE, axis=1) - one
    o3[...] = tmpE * tmpE
    tmpBCH = uniqBCHE * (one - uniqBCHE)
    tmpBCT = uniqBCTE * (one - uniqBCTE)
    o4[...] = (jnp.sum(tmpBCH * tmpBCH, axis=1)
               + jnp.sum(tmpBCT * tmpBCT, axis=1))
    tmpBCHL = jnp.maximum(one - jnp.sum(jnp.abs(uniqBCHE), axis=1), 0.0)
    tmpBCTL = jnp.maximum(one - jnp.sum(jnp.abs(uniqBCTE), axis=1), 0.0)
    o5[...] = tmpBCHL + tmpBCTL
    o6[...] = jnp.maximum(m + tmpTransE - tmpNTransE, 0.0)


def _tc_compute(margin2d, gathered):
    vec = jax.ShapeDtypeStruct((B,), jnp.float32)
    return pl.pallas_call(
        _tc_body,
        grid=(B // _R,),
        in_specs=[
            pl.BlockSpec((1, 1), lambda i: (0, 0)),
            pl.BlockSpec((NG, _R, D), lambda i: (0, i, 0)),
        ],
        out_specs=[pl.BlockSpec((_R,), lambda i: (i,))] * 6,
        out_shape=[vec] * 6,
    )(margin2d, gathered)


def kernel(aBHE, aBTE, aBC, aHead, aTail, aRelation, nABHE, nABTE, nABC,
           nHead, nTail, nRelation, uniqE, uniqBC, lossMargin, device,
           entityEmbed, bConceptHEmbed, bConceptTEmbed, headEmbed,
           tailEmbed, relationEmbed):
    idx_all = jnp.stack([
        aBHE, aBTE, aBC, aHead, aTail, aRelation,
        nABHE, nABTE, nABC, nHead, nTail, nRelation, uniqE, uniqBC,
    ]).astype(jnp.int32)
    gathered = _sc_gather(entityEmbed, bConceptHEmbed, bConceptTEmbed,
                          headEmbed, tailEmbed, relationEmbed, idx_all)
    margin2d = jnp.asarray(lossMargin, jnp.float32).reshape(1, 1)
    o = _tc_compute(margin2d, gathered)
    return (o[0], o[1], o[2], o[3], o[4], o[5])


# speedup vs baseline: 1.3880x; 1.3880x over previous
import functools

import jax
import jax.numpy as jnp
from jax import lax
from jax.experimental import pallas as pl
from jax.experimental.pallas import tpu as pltpu
from jax.experimental.pallas import tpu_sc as plsc

D = 64
N = 100000
B = 16384
NC = 2
NS = 16
NW = NC * NS
PER_W = B // NW
CH = 128
NCH = PER_W // CH

NIDX = 14

NG = 17

_GATHERS = (
    (0, 2, ((2, 0), (3, 1))),
    (0, 8, ((9, 0), (10, 1))),
    (0, 13, ((15, 0), (16, 1))),
    (1, 0, ((0, 0),)),
    (1, 1, ((1, 0),)),
    (1, 6, ((7, 0),)),
    (1, 7, ((8, 0),)),
    (1, 12, ((14, 0),)),
    (1, 4, ((5, 1),)),
    (1, 10, ((12, 1),)),
    (2, 3, ((4, 0),)),
    (2, 9, ((11, 0),)),
    (2, 5, ((6, 1),)),
    (2, 11, ((13, 1),)),
)

_MESH = plsc.VectorSubcoreMesh(
    core_axis_name="c", subcore_axis_name="s", num_cores=NC, num_subcores=NS
)


_EC = 2048
_NEB = (N + _EC - 1) // _EC


def _repack_body(entT, bchT, bctT, headT, tailT, relT, p0, p1, p2):
    p0[:, :D] = bchT[...].T
    p0[:, D:] = bctT[...].T
    p1[:, :D] = entT[...].T
    p1[:, D:] = tailT[...].T
    p2[:, :D] = headT[...].T
    p2[:, D:] = relT[...].T


def _repack(entT, bchT, bctT, headT, tailT, relT):
    pair = jax.ShapeDtypeStruct((N, 2 * D), jnp.float32)
    in_spec = pl.BlockSpec((D, _EC), lambda i: (0, i))
    out_spec = pl.BlockSpec((_EC, 2 * D), lambda i: (i, 0))
    return pl.pallas_call(
        _repack_body,
        grid=(_NEB,),
        in_specs=[in_spec] * 6,
        out_specs=[out_spec] * 3,
        out_shape=[pair] * 3,
        compiler_params=pltpu.CompilerParams(
            dimension_semantics=("arbitrary",),
        ),
    )(entT, bchT, bctT, headT, tailT, relT)


@functools.partial(
    pl.kernel,
    out_type=jax.ShapeDtypeStruct((NG, B, D), jnp.float32),
    mesh=_MESH,
    scratch_types=[
        pltpu.VMEM((NIDX, PER_W), jnp.int32),
        pltpu.VMEM((2, CH, 2 * D), jnp.float32),
        pltpu.SemaphoreType.DMA,
        pltpu.SemaphoreType.DMA,
        pltpu.SemaphoreType.DMA,
    ],
    compiler_params=pltpu.CompilerParams(use_tc_tiling_on_sc=False),
)
def _sc_gather(p0, p1, p2, idx_hbm, out, idx_v, rows, sem_i, sem_a, sem_b):
    pairs = (p0, p1, p2)
    wid = lax.axis_index("s") * NC + lax.axis_index("c")
    base = wid * PER_W
    pltpu.async_copy(idx_hbm.at[:, pl.ds(base, PER_W)], idx_v, sem_i).wait()
    for pi, ii, outs in _GATHERS:
        tbl = pairs[pi]

        @pl.loop(0, NCH, step=2)
        def _(c, tbl=tbl, ii=ii, outs=outs):
            o0 = c * CH
            o1 = o0 + CH
            cp_a = pltpu.async_copy(
                tbl.at[idx_v.at[ii, pl.ds(o0, CH)]], rows.at[0], sem_a)
            cp_b = pltpu.async_copy(
                tbl.at[idx_v.at[ii, pl.ds(o1, CH)]], rows.at[1], sem_b)
            cp_a.wait()
            for g, h in outs:
                pltpu.sync_copy(rows.at[0, :, pl.ds(h * D, D)],
                                out.at[g, pl.ds(base + o0, CH)])
            cp_b.wait()
            for g, h in outs:
                pltpu.sync_copy(rows.at[1, :, pl.ds(h * D, D)],
                                out.at[g, pl.ds(base + o1, CH)])


_R = 1024


def _tc_body(m_ref, g_ref, o1, o2, o3, o4, o5, o6):
    m = m_ref[0, 0]
    one = jnp.float32(1.0)
    x = g_ref[...]
    aBHEE, aBTEE, aBCHE, aBCTE = x[0], x[1], x[2], x[3]
    aHeadE, aTailE, aRelE = x[4], x[5], x[6]
    nABHEE, nABTEE, nABCHE, nABCTE = x[7], x[8], x[9], x[10]
    nHeadE, nTailE, nRelE = x[11], x[12], x[13]
    uniqEE, uniqBCHE, uniqBCTE = x[14], x[15], x[16]

    tmpBE2CH = (one - aBCHE) * aBHEE
    tmpBE2CT = (one - aBCTE) * aBTEE
    tmpTransE = jnp.sum(jnp.abs(aHeadE + aRelE - aTailE), axis=1)
    o1[...] = (jnp.sum(tmpBE2CH * tmpBE2CH, axis=1)
               + jnp.sum(tmpBE2CT * tmpBE2CT, axis=1) + tmpTransE)
    tmpNBE2CH = (one - nABCHE) * nABHEE
    tmpNBE2CT = (one - nABCTE) * nABTEE
    tmpNTransE = jnp.sum(jnp.abs(nHeadE + nRelE - nTailE), axis=1)
    tmpNBL = (jnp.sum(tmpNBE2CH * tmpNBE2CH, axis=1)
              + jnp.sum(tmpNBE2CT * tmpNBE2CT, axis=1) + tmpNTransE)
    o2[...] = jnp.maximum(m - tmpNBL, 0.0)
    tmpE = jnp.sum(uniqEE * uniqEE, axis=1) - one
    o3[...] = tmpE * tmpE
    tmpBCH = uniqBCHE * (one - uniqBCHE)
    tmpBCT = uniqBCTE * (one - uniqBCTE)
    o4[...] = (jnp.sum(tmpBCH * tmpBCH, axis=1)
               + jnp.sum(tmpBCT * tmpBCT, axis=1))
    tmpBCHL = jnp.maximum(one - jnp.sum(jnp.abs(uniqBCHE), axis=1), 0.0)
    tmpBCTL = jnp.maximum(one - jnp.sum(jnp.abs(uniqBCTE), axis=1), 0.0)
    o5[...] = tmpBCHL + tmpBCTL
    o6[...] = jnp.maximum(m + tmpTransE - tmpNTransE, 0.0)


def _tc_compute(margin2d, gathered):
    vec = jax.ShapeDtypeStruct((B,), jnp.float32)
    return pl.pallas_call(
        _tc_body,
        grid=(B // _R,),
        in_specs=[
            pl.BlockSpec((1, 1), lambda i: (0, 0)),
            pl.BlockSpec((NG, _R, D), lambda i: (0, i, 0)),
        ],
        out_specs=[pl.BlockSpec((_R,), lambda i: (i,))] * 6,
        out_shape=[vec] * 6,
    )(margin2d, gathered)


def kernel(aBHE, aBTE, aBC, aHead, aTail, aRelation, nABHE, nABTE, nABC,
           nHead, nTail, nRelation, uniqE, uniqBC, lossMargin, device,
           entityEmbed, bConceptHEmbed, bConceptTEmbed, headEmbed,
           tailEmbed, relationEmbed):
    idx_all = jnp.stack([
        aBHE, aBTE, aBC, aHead, aTail, aRelation,
        nABHE, nABTE, nABC, nHead, nTail, nRelation, uniqE, uniqBC,
    ]).astype(jnp.int32)
    sw = lambda t: jnp.swapaxes(t, 0, 1)
    p0, p1, p2 = _repack(sw(entityEmbed), sw(bConceptHEmbed),
                         sw(bConceptTEmbed), sw(headEmbed),
                         sw(tailEmbed), sw(relationEmbed))
    gathered = _sc_gather(p0, p1, p2, idx_all)
    margin2d = jnp.asarray(lossMargin, jnp.float32).reshape(1, 1)
    o = _tc_compute(margin2d, gathered)
    return (o[0], o[1], o[2], o[3], o[4], o[5])


# speedup vs baseline: 1.4908x; 1.0741x over previous
import functools

import jax
import jax.numpy as jnp
from jax import lax
from jax.experimental import pallas as pl
from jax.experimental.pallas import tpu as pltpu
from jax.experimental.pallas import tpu_sc as plsc

D = 64
N = 100000
B = 16384
NC = 2
NS = 16
NW = NC * NS
PER_W = B // NW
CH = 128
NCH = PER_W // CH

NIDX = 14

NG = 17

_GATHERS = (
    (0, 2, ((2, 0), (3, 1))),
    (0, 8, ((9, 0), (10, 1))),
    (0, 13, ((15, 0), (16, 1))),
    (1, 0, ((0, 0),)),
    (1, 1, ((1, 0),)),
    (1, 6, ((7, 0),)),
    (1, 7, ((8, 0),)),
    (1, 12, ((14, 0),)),
    (1, 4, ((5, 1),)),
    (1, 10, ((12, 1),)),
    (2, 3, ((4, 0),)),
    (2, 9, ((11, 0),)),
    (2, 5, ((6, 1),)),
    (2, 11, ((13, 1),)),
)

_MESH = plsc.VectorSubcoreMesh(
    core_axis_name="c", subcore_axis_name="s", num_cores=NC, num_subcores=NS
)


_EC = 2048
_NEB = (N + _EC - 1) // _EC


def _repack_body(entT, bchT, bctT, headT, tailT, relT, p0, p1, p2):
    p0[:, :D] = bchT[...].T
    p0[:, D:] = bctT[...].T
    p1[:, :D] = entT[...].T
    p1[:, D:] = tailT[...].T
    p2[:, :D] = headT[...].T
    p2[:, D:] = relT[...].T


def _repack(entT, bchT, bctT, headT, tailT, relT):
    pair = jax.ShapeDtypeStruct((N, 2 * D), jnp.float32)
    in_spec = pl.BlockSpec((D, _EC), lambda i: (0, i))
    out_spec = pl.BlockSpec((_EC, 2 * D), lambda i: (i, 0))
    return pl.pallas_call(
        _repack_body,
        grid=(_NEB,),
        in_specs=[in_spec] * 6,
        out_specs=[out_spec] * 3,
        out_shape=[pair] * 3,
        compiler_params=pltpu.CompilerParams(
            dimension_semantics=("parallel",),
        ),
    )(entT, bchT, bctT, headT, tailT, relT)


@functools.partial(
    pl.kernel,
    out_type=jax.ShapeDtypeStruct((NG, B, D), jnp.float32),
    mesh=_MESH,
    scratch_types=[
        pltpu.VMEM((NIDX, PER_W), jnp.int32),
        pltpu.VMEM((2, CH, 2 * D), jnp.float32),
        pltpu.SemaphoreType.DMA,
        pltpu.SemaphoreType.DMA,
        pltpu.SemaphoreType.DMA,
    ],
    compiler_params=pltpu.CompilerParams(use_tc_tiling_on_sc=False),
)
def _sc_gather(p0, p1, p2, idx_hbm, out, idx_v, rows, sem_i, sem_a, sem_b):
    pairs = (p0, p1, p2)
    wid = lax.axis_index("s") * NC + lax.axis_index("c")
    base = wid * PER_W
    pltpu.async_copy(idx_hbm.at[:, pl.ds(base, PER_W)], idx_v, sem_i).wait()
    for pi, ii, outs in _GATHERS:
        tbl = pairs[pi]

        @pl.loop(0, NCH, step=2)
        def _(c, tbl=tbl, ii=ii, outs=outs):
            o0 = c * CH
            o1 = o0 + CH
            cp_a = pltpu.async_copy(
                tbl.at[idx_v.at[ii, pl.ds(o0, CH)]], rows.at[0], sem_a)
            cp_b = pltpu.async_copy(
                tbl.at[idx_v.at[ii, pl.ds(o1, CH)]], rows.at[1], sem_b)
            cp_a.wait()
            for g, h in outs:
                pltpu.sync_copy(rows.at[0, :, pl.ds(h * D, D)],
                                out.at[g, pl.ds(base + o0, CH)])
            cp_b.wait()
            for g, h in outs:
                pltpu.sync_copy(rows.at[1, :, pl.ds(h * D, D)],
                                out.at[g, pl.ds(base + o1, CH)])


_R = 1024
_RH = _R // 2


def _tc_body(m_ref, g_ref, o1, o2, o3, o4, o5, o6):
    m = m_ref[0, 0]
    one = jnp.float32(1.0)
    x = g_ref[...]

    def rsum(v):
        s = jnp.sum(v.reshape(_RH, 2, D), axis=2)
        return s.reshape(_R)

    aBHEE, aBTEE, aBCHE, aBCTE = x[0], x[1], x[2], x[3]
    aHeadE, aTailE, aRelE = x[4], x[5], x[6]
    nABHEE, nABTEE, nABCHE, nABCTE = x[7], x[8], x[9], x[10]
    nHeadE, nTailE, nRelE = x[11], x[12], x[13]
    uniqEE, uniqBCHE, uniqBCTE = x[14], x[15], x[16]

    tmpBE2CH = (one - aBCHE) * aBHEE
    tmpBE2CT = (one - aBCTE) * aBTEE
    tmpTransE = rsum(jnp.abs(aHeadE + aRelE - aTailE))
    o1[...] = (rsum(tmpBE2CH * tmpBE2CH)
               + rsum(tmpBE2CT * tmpBE2CT) + tmpTransE)
    tmpNBE2CH = (one - nABCHE) * nABHEE
    tmpNBE2CT = (one - nABCTE) * nABTEE
    tmpNTransE = rsum(jnp.abs(nHeadE + nRelE - nTailE))
    tmpNBL = (rsum(tmpNBE2CH * tmpNBE2CH)
              + rsum(tmpNBE2CT * tmpNBE2CT) + tmpNTransE)
    o2[...] = jnp.maximum(m - tmpNBL, 0.0)
    tmpE = rsum(uniqEE * uniqEE) - one
    o3[...] = tmpE * tmpE
    tmpBCH = uniqBCHE * (one - uniqBCHE)
    tmpBCT = uniqBCTE * (one - uniqBCTE)
    o4[...] = rsum(tmpBCH * tmpBCH) + rsum(tmpBCT * tmpBCT)
    o5[...] = (jnp.maximum(one - rsum(jnp.abs(uniqBCHE)), 0.0)
               + jnp.maximum(one - rsum(jnp.abs(uniqBCTE)), 0.0))
    o6[...] = jnp.maximum(m + tmpTransE - tmpNTransE, 0.0)


def _tc_compute(margin2d, gathered_packed):
    vec = jax.ShapeDtypeStruct((B,), jnp.float32)
    return pl.pallas_call(
        _tc_body,
        grid=(B // _R,),
        in_specs=[
            pl.BlockSpec((1, 1), lambda i: (0, 0)),
            pl.BlockSpec((NG, _RH, 2 * D), lambda i: (0, i, 0)),
        ],
        out_specs=[pl.BlockSpec((_R,), lambda i: (i,))] * 6,
        out_shape=[vec] * 6,
        compiler_params=pltpu.CompilerParams(
            dimension_semantics=("parallel",),
        ),
    )(margin2d, gathered_packed)


def kernel(aBHE, aBTE, aBC, aHead, aTail, aRelation, nABHE, nABTE, nABC,
           nHead, nTail, nRelation, uniqE, uniqBC, lossMargin, device,
           entityEmbed, bConceptHEmbed, bConceptTEmbed, headEmbed,
           tailEmbed, relationEmbed):
    idx_all = jnp.stack([
        aBHE, aBTE, aBC, aHead, aTail, aRelation,
        nABHE, nABTE, nABC, nHead, nTail, nRelation, uniqE, uniqBC,
    ]).astype(jnp.int32)
    sw = lambda t: jnp.swapaxes(t, 0, 1)
    p0, p1, p2 = _repack(sw(entityEmbed), sw(bConceptHEmbed),
                         sw(bConceptTEmbed), sw(headEmbed),
                         sw(tailEmbed), sw(relationEmbed))
    gathered = _sc_gather(p0, p1, p2, idx_all)
    gathered_packed = jnp.reshape(gathered, (NG, B // 2, 2 * D))
    margin2d = jnp.asarray(lossMargin, jnp.float32).reshape(1, 1)
    o = _tc_compute(margin2d, gathered_packed)
    return (o[0], o[1], o[2], o[3], o[4], o[5])


# speedup vs baseline: 1.7655x; 1.1843x over previous
import functools

import jax
import jax.numpy as jnp
from jax import lax
from jax.experimental import pallas as pl
from jax.experimental.pallas import tpu as pltpu
from jax.experimental.pallas import tpu_sc as plsc

D = 64
N = 100000
B = 16384
BH = B // 2
NC = 2
NS = 16
NW = NC * NS
PER_W = B // NW
CH = 128
NCH = PER_W // CH

_MESH = plsc.VectorSubcoreMesh(
    core_axis_name="c", subcore_axis_name="s", num_cores=NC, num_subcores=NS
)

_G_ET = (
    (0, ((0, 0),)),
    (1, ((1, 0),)),
    (2, ((2, 0),)),
    (3, ((3, 0),)),
    (4, ((4, 0),)),
    (5, ((5, 1),)),
    (6, ((6, 1),)),
)
_G_BC = (
    (0, ((0, 0), (1, 1))),
    (1, ((2, 0), (3, 1))),
    (2, ((4, 0), (5, 1))),
)
_G_HR = (
    (0, ((0, 0),)),
    (1, ((1, 0),)),
    (2, ((2, 1),)),
    (3, ((3, 1),)),
)


_EC = 2048
_NEB = (N + _EC - 1) // _EC


def _repack_body(aT, bT, p):
    p[:, :D] = aT[...].T
    p[:, D:] = bT[...].T


def _repack(aT, bT):
    return pl.pallas_call(
        _repack_body,
        grid=(_NEB,),
        in_specs=[pl.BlockSpec((D, _EC), lambda i: (0, i))] * 2,
        out_specs=pl.BlockSpec((_EC, 2 * D), lambda i: (i, 0)),
        out_shape=jax.ShapeDtypeStruct((N, 2 * D), jnp.float32),
        compiler_params=pltpu.CompilerParams(
            dimension_semantics=("arbitrary",),
        ),
    )(aT, bT)


def _make_sc_gather(gathers, n_idx, n_out):
    @functools.partial(
        pl.kernel,
        out_type=jax.ShapeDtypeStruct((n_out, BH, 2 * D), jnp.float32),
        mesh=_MESH,
        scratch_types=[
            pltpu.VMEM((n_idx, PER_W), jnp.int32),
            pltpu.VMEM((2, CH, 2 * D), jnp.float32),
            pltpu.SemaphoreType.DMA,
            pltpu.SemaphoreType.DMA,
            pltpu.SemaphoreType.DMA,
        ],
        compiler_params=pltpu.CompilerParams(use_tc_tiling_on_sc=False),
    )
    def sc_gather(tbl, idx_hbm, out, idx_v, rows, sem_i, sem_a, sem_b):
        wid = lax.axis_index("s") * NC + lax.axis_index("c")
        base = wid * PER_W
        half = base // BH
        row0 = base - half * BH
        pltpu.async_copy(idx_hbm.at[:, pl.ds(base, PER_W)], idx_v, sem_i
                         ).wait()
        for ii, outs in gathers:

            @pl.loop(0, NCH, step=2)
            def _(c, ii=ii, outs=outs):
                o0 = c * CH
                o1 = o0 + CH
                cp_a = pltpu.async_copy(
                    tbl.at[idx_v.at[ii, pl.ds(o0, CH)]], rows.at[0], sem_a)
                cp_b = pltpu.async_copy(
                    tbl.at[idx_v.at[ii, pl.ds(o1, CH)]], rows.at[1], sem_b)
                cp_a.wait()
                for g, h in outs:
                    pltpu.sync_copy(
                        rows.at[0, :, pl.ds(h * D, D)],
                        out.at[g, pl.ds(row0 + o0, CH),
                               pl.ds(half * D, D)])
                cp_b.wait()
                for g, h in outs:
                    pltpu.sync_copy(
                        rows.at[1, :, pl.ds(h * D, D)],
                        out.at[g, pl.ds(row0 + o1, CH),
                               pl.ds(half * D, D)])

    return sc_gather


_sc_et = _make_sc_gather(_G_ET, 7, 7)
_sc_bc = _make_sc_gather(_G_BC, 3, 6)
_sc_hr = _make_sc_gather(_G_HR, 4, 4)


_RH = 512


def _tc_body(m_ref, et_ref, bc_ref, hr_ref, o1, o2, o3, o4, o5, o6):
    m = m_ref[0, 0]
    one = jnp.float32(1.0)
    et = et_ref[...]
    bc = bc_ref[...]
    hr = hr_ref[...]

    for h in (0, 1):
        sl = slice(h * D, h * D + D)
        aBHEE, aBTEE = et[0][:, sl], et[1][:, sl]
        nABHEE, nABTEE = et[2][:, sl], et[3][:, sl]
        uniqEE = et[4][:, sl]
        aTailE, nTailE = et[5][:, sl], et[6][:, sl]
        aBCHE, aBCTE = bc[0][:, sl], bc[1][:, sl]
        nABCHE, nABCTE = bc[2][:, sl], bc[3][:, sl]
        uniqBCHE, uniqBCTE = bc[4][:, sl], bc[5][:, sl]
        aHeadE, nHeadE = hr[0][:, sl], hr[1][:, sl]
        aRelE, nRelE = hr[2][:, sl], hr[3][:, sl]

        rs = lambda v: jnp.sum(v, axis=1)
        tmpBE2CH = (one - aBCHE) * aBHEE
        tmpBE2CT = (one - aBCTE) * aBTEE
        tmpTransE = rs(jnp.abs(aHeadE + aRelE - aTailE))
        o1[h, :] = (rs(tmpBE2CH * tmpBE2CH) + rs(tmpBE2CT * tmpBE2CT)
                    + tmpTransE)
        tmpNBE2CH = (one - nABCHE) * nABHEE
        tmpNBE2CT = (one - nABCTE) * nABTEE
        tmpNTransE = rs(jnp.abs(nHeadE + nRelE - nTailE))
        tmpNBL = (rs(tmpNBE2CH * tmpNBE2CH) + rs(tmpNBE2CT * tmpNBE2CT)
                  + tmpNTransE)
        o2[h, :] = jnp.maximum(m - tmpNBL, 0.0)
        tmpE = rs(uniqEE * uniqEE) - one
        o3[h, :] = tmpE * tmpE
        tmpBCH = uniqBCHE * (one - uniqBCHE)
        tmpBCT = uniqBCTE * (one - uniqBCTE)
        o4[h, :] = rs(tmpBCH * tmpBCH) + rs(tmpBCT * tmpBCT)
        o5[h, :] = (jnp.maximum(one - rs(jnp.abs(uniqBCHE)), 0.0)
                    + jnp.maximum(one - rs(jnp.abs(uniqBCTE)), 0.0))
        o6[h, :] = jnp.maximum(m + tmpTransE - tmpNTransE, 0.0)


def _tc_compute(margin2d, g_et, g_bc, g_hr):
    halves = jax.ShapeDtypeStruct((2, BH), jnp.float32)
    blk = lambda n: pl.BlockSpec((n, _RH, 2 * D), lambda i: (0, i, 0))
    return pl.pallas_call(
        _tc_body,
        grid=(BH // _RH,),
        in_specs=[
            pl.BlockSpec((1, 1), lambda i: (0, 0)),
            blk(7), blk(6), blk(4),
        ],
        out_specs=[pl.BlockSpec((2, _RH), lambda i: (0, i))] * 6,
        out_shape=[halves] * 6,
        compiler_params=pltpu.CompilerParams(
            dimension_semantics=("arbitrary",),
        ),
    )(margin2d, g_et, g_bc, g_hr)


def kernel(aBHE, aBTE, aBC, aHead, aTail, aRelation, nABHE, nABTE, nABC,
           nHead, nTail, nRelation, uniqE, uniqBC, lossMargin, device,
           entityEmbed, bConceptHEmbed, bConceptTEmbed, headEmbed,
           tailEmbed, relationEmbed):
    i32 = lambda a: a.astype(jnp.int32)
    idx_et = jnp.stack([i32(aBHE), i32(aBTE), i32(nABHE), i32(nABTE),
                        i32(uniqE), i32(aTail), i32(nTail)])
    idx_bc = jnp.stack([i32(aBC), i32(nABC), i32(uniqBC)])
    idx_hr = jnp.stack([i32(aHead), i32(nHead), i32(aRelation),
                        i32(nRelation)])
    sw = lambda t: jnp.swapaxes(t, 0, 1)
    p_et = _repack(sw(entityEmbed), sw(tailEmbed))
    g_et = _sc_et(p_et, idx_et)
    p_bc = _repack(sw(bConceptHEmbed), sw(bConceptTEmbed))
    g_bc = _sc_bc(p_bc, idx_bc)
    p_hr = _repack(sw(headEmbed), sw(relationEmbed))
    g_hr = _sc_hr(p_hr, idx_hr)
    margin2d = jnp.asarray(lossMargin, jnp.float32).reshape(1, 1)
    o = _tc_compute(margin2d, g_et, g_bc, g_hr)
    join = lambda t: jnp.concatenate([t[0], t[1]])
    return tuple(join(t) for t in o)
